# baseline (speedup 1.0000x reference)
import functools
import numpy as np
import jax
import jax.numpy as jnp
from jax import lax
from jax.experimental import pallas as pl
from jax.experimental.pallas import tpu as pltpu
from jax.experimental.pallas import tpu_sc as plsc

VOCAB = 100000
CUTS = (0, 2000, 10000, 50000, 100000)
CUT1, CUT2, CUT3 = 2000, 10000, 50000
H = 768
LPAD = 2048
TT = 256
NTT = LPAD // TT
VT = 1024
NVT = (VOCAB + VT - 1) // VT
MAX_ITEMS = NVT * NTT

_c_lo = np.array([int(np.searchsorted(CUTS, v * VT, 'right') - 1)
                  for v in range(NVT)], np.int32)
_c_hi = np.array([int(np.searchsorted(CUTS, min((v + 1) * VT, VOCAB) - 1,
                                      'right') - 1)
                  for v in range(NVT)], np.int32)

_SC_NW = 32
_BPW = LPAD // _SC_NW


def _cluster_of(v):
    return ((v >= CUT1).astype(jnp.int32) + (v >= CUT2).astype(jnp.int32)
            + (v >= CUT3).astype(jnp.int32))


def _schedule(yf):
    n = yf.shape[0]
    cl = _cluster_of(yf)
    clp = jnp.concatenate(
        [cl, jnp.full((LPAD - n,), 4, jnp.int32)])
    perm = jnp.argsort(clp, stable=True).astype(jnp.int32)
    cls = jnp.sort(clp)
    offs = jnp.searchsorted(cls, jnp.arange(5, dtype=jnp.int32),
                            side='left').astype(jnp.int32)
    start = offs[_c_lo]
    end = offs[_c_hi + 1]
    tlo = (start // TT).astype(jnp.int32)
    cnt = jnp.where(end > start,
                    (end + TT - 1) // TT - start // TT, 0).astype(jnp.int32)
    csum = jnp.cumsum(cnt)
    base = jnp.concatenate([jnp.zeros((1,), jnp.int32),
                            csum[:-1].astype(jnp.int32)])
    total = csum[-1]
    j = jnp.arange(MAX_ITEMS, dtype=jnp.int32)
    vj = jnp.clip(jnp.searchsorted(base, j, side='right') - 1,
                  0, NVT - 1).astype(jnp.int32)
    tt = jnp.clip(tlo[vj] + (j - base[vj]), 0, NTT - 1).astype(jnp.int32)
    valid = (j < total).astype(jnp.int32)
    return perm, tt, vj, valid


def _grouped_body(tt_ref, wt_ref, valid_ref, y_ref, x_ref, w_ref, b_ref,
                  cw_ref, cb_ref, out_ref, s_acc, t_acc, cll):
    j = pl.program_id(0)

    @pl.when(j == 0)
    def _init():
        s_acc[:] = jnp.zeros_like(s_acc)
        t_acc[:] = jnp.zeros_like(t_acc)
        clg = jnp.dot(x_ref[:], cw_ref[:],
                      preferred_element_type=jnp.float32) + cb_ref[:]
        m = jnp.max(clg, axis=1, keepdims=True)
        lse = m + jnp.log(jnp.sum(jnp.exp(clg - m), axis=1, keepdims=True))
        ccol = jax.lax.broadcasted_iota(jnp.int32, (1, clg.shape[1]), 1)
        tok_cl = _cluster_of(y_ref[:])
        cll[:] = jnp.sum(jnp.where(ccol == tok_cl, clg - lse, 0.0),
                         axis=1, keepdims=True)

    @pl.when(valid_ref[j] != 0)
    def _item():
        r0 = tt_ref[j] * TT
        wt = wt_ref[j]
        xt = x_ref[pl.ds(r0, TT), :]
        logits = jnp.dot(xt, w_ref[:],
                         preferred_element_type=jnp.float32) + b_ref[:]
        col = wt * VT + jax.lax.broadcasted_iota(jnp.int32, (1, VT), 1)
        col_cl = jnp.where(col < VOCAB, _cluster_of(col), -1)
        yt = y_ref[pl.ds(r0, TT), :]
        tok_cl = _cluster_of(yt)
        s_acc[pl.ds(r0, TT), :] += jnp.sum(
            jnp.where(col_cl == tok_cl, jnp.exp(logits), 0.0),
            axis=1, keepdims=True)
        t_acc[pl.ds(r0, TT), :] += jnp.sum(
            jnp.where(col == yt, logits, 0.0), axis=1, keepdims=True)

    @pl.when(j == MAX_ITEMS - 1)
    def _finish():
        nll = -(cll[:] + t_acc[:] - jnp.log(s_acc[:]))
        out_ref[:] = jnp.broadcast_to(nll, (LPAD, 128))


def _tc_grouped(x_s, y_s, W, b, cW, cb, tt, wt, valid):
    grid_spec = pltpu.PrefetchScalarGridSpec(
        num_scalar_prefetch=3,
        grid=(MAX_ITEMS,),
        in_specs=[
            pl.BlockSpec((LPAD, 1), lambda j, t, w, v: (0, 0)),
            pl.BlockSpec((LPAD, H), lambda j, t, w, v: (0, 0)),
            pl.BlockSpec((H, VT), lambda j, t, w, v: (0, w[j])),
            pl.BlockSpec((1, VT), lambda j, t, w, v: (0, w[j])),
            pl.BlockSpec(cW.shape, lambda j, t, w, v: (0, 0)),
            pl.BlockSpec(cb.shape, lambda j, t, w, v: (0, 0)),
        ],
        out_specs=pl.BlockSpec((LPAD, 128), lambda j, t, w, v: (0, 0)),
        scratch_shapes=[
            pltpu.VMEM((LPAD, 1), jnp.float32),
            pltpu.VMEM((LPAD, 1), jnp.float32),
            pltpu.VMEM((LPAD, 1), jnp.float32),
        ],
    )
    return pl.pallas_call(
        _grouped_body,
        grid_spec=grid_spec,
        out_shape=jax.ShapeDtypeStruct((LPAD, 128), jnp.float32),
        compiler_params=pltpu.CompilerParams(
            dimension_semantics=("arbitrary",)),
    )(tt, wt, valid, y_s, x_s, W, b, cW, cb)


def _sc_gather_rows(x_pad, perm):
    mesh = plsc.VectorSubcoreMesh(core_axis_name="c", subcore_axis_name="s")

    @functools.partial(
        pl.kernel, mesh=mesh,
        out_type=jax.ShapeDtypeStruct((LPAD, H), jnp.float32),
        scratch_types=[
            pltpu.VMEM((_BPW,), jnp.int32),
            pltpu.VMEM((_BPW, H), jnp.float32),
            pltpu.SemaphoreType.DMA,
        ],
    )
    def k(x_hbm, idx_hbm, out_hbm, idx_v, rows_v, sem):
        wid = lax.axis_index("s") * 2 + lax.axis_index("c")
        b0 = wid * _BPW
        pltpu.sync_copy(idx_hbm.at[pl.ds(b0, _BPW)], idx_v)
        pltpu.async_copy(x_hbm.at[idx_v], rows_v, sem).wait()
        pltpu.sync_copy(rows_v, out_hbm.at[pl.ds(b0, _BPW)])

    return k(x_pad, perm)


def _sc_scatter_rows(src, perm):
    mesh = plsc.VectorSubcoreMesh(core_axis_name="c", subcore_axis_name="s")

    @functools.partial(
        pl.kernel, mesh=mesh,
        out_type=jax.ShapeDtypeStruct((LPAD, 128), jnp.float32),
        scratch_types=[
            pltpu.VMEM((_BPW,), jnp.int32),
            pltpu.VMEM((_BPW, 128), jnp.float32),
            pltpu.SemaphoreType.DMA,
        ],
    )
    def k(src_hbm, idx_hbm, out_hbm, idx_v, rows_v, sem):
        wid = lax.axis_index("s") * 2 + lax.axis_index("c")
        b0 = wid * _BPW
        pltpu.sync_copy(idx_hbm.at[pl.ds(b0, _BPW)], idx_v)
        pltpu.sync_copy(src_hbm.at[pl.ds(b0, _BPW)], rows_v)
        pltpu.async_copy(rows_v, out_hbm.at[idx_v], sem).wait()

    return k(src, perm)


def kernel(x, y, cluster_W, cluster_b, W, b):
    x = x[:, :-1]
    bsz, l, h = x.shape
    xf = x.reshape(bsz * l, h)
    yf = y.reshape(-1)
    n = xf.shape[0]
    xp = jnp.pad(xf, ((0, LPAD - n), (0, 0)))
    yp = jnp.pad(yf, (0, LPAD - n), constant_values=-1)

    perm, tt, wt, valid = _schedule(yf)
    x_s = _sc_gather_rows(xp, perm)
    y_s = yp[perm].reshape(LPAD, 1)
    nll_s = _tc_grouped(x_s, y_s, W, b, cluster_W, cluster_b, tt, wt, valid)
    nll = _sc_scatter_rows(nll_s, perm)
    return nll[:n, 0]
